# baseline (speedup 1.0000x reference)
import functools

import jax
import jax.numpy as jnp
from jax import lax
from jax.experimental import pallas as pl
from jax.experimental.pallas import tpu as pltpu
from jax.experimental.pallas import tpu_sc as plsc

N_NODES_C = 100000
N_EDGES_C = 6400000

NCORES = 2
NSUB = 16
NW = NCORES * NSUB

EB = 2048
NB = N_EDGES_C // EB
NB_PER_W = -(-NB // NW)

NPAD = 100352
SLICE = NPAD // NSUB

_mesh = plsc.VectorSubcoreMesh(core_axis_name="c", subcore_axis_name="s")
_sc_params = pltpu.CompilerParams(use_tc_tiling_on_sc=False)

f32 = jnp.float32
i32 = jnp.int32


@functools.partial(
    pl.kernel,
    mesh=_mesh,
    compiler_params=_sc_params,
    out_type=[
        jax.ShapeDtypeStruct((NCORES, NPAD), f32),
        jax.ShapeDtypeStruct((NCORES, NPAD), f32),
    ],
    scratch_types=[
        pltpu.VMEM((1, EB), i32),
        pltpu.VMEM((1, EB), i32),
        pltpu.VMEM((EB,), f32),
        pltpu.VMEM((SLICE,), f32),
        pltpu.VMEM_SHARED((NPAD,), f32),
        pltpu.VMEM_SHARED((NPAD,), f32),
    ],
)
def _sc_degrees(src_hbm, dst_hbm, dego_hbm, degi_hbm,
                sbuf, dbuf, ones_v, bounce, dego_sh, degi_sh):
    c = lax.axis_index("c")
    s = lax.axis_index("s")
    wid = c * NSUB + s

    @pl.loop(0, EB, step=16)
    def _(i):
        ones_v[pl.ds(i, 16)] = jnp.ones((16,), f32)

    @pl.loop(0, SLICE, step=16)
    def _(i):
        bounce[pl.ds(i, 16)] = jnp.zeros((16,), f32)

    sl = pl.ds(s * SLICE, SLICE)
    pltpu.sync_copy(bounce, dego_sh.at[sl])
    pltpu.sync_copy(bounce, degi_sh.at[sl])
    plsc.subcore_barrier()

    @pl.loop(0, NB_PER_W)
    def _(i):
        b = wid + i * NW

        @pl.when(b < NB)
        def _():
            pltpu.sync_copy(src_hbm.at[pl.ds(b, 1)], sbuf)
            pltpu.sync_copy(dst_hbm.at[pl.ds(b, 1)], dbuf)
            pltpu.sync_copy(ones_v, dego_sh.at[sbuf.at[0]], add=True)
            pltpu.sync_copy(ones_v, degi_sh.at[dbuf.at[0]], add=True)

    plsc.subcore_barrier()
    pltpu.sync_copy(dego_sh.at[sl], bounce)
    pltpu.sync_copy(bounce, dego_hbm.at[c].at[sl])
    pltpu.sync_copy(degi_sh.at[sl], bounce)
    pltpu.sync_copy(bounce, degi_hbm.at[c].at[sl])


def _make_msg_pass(C):
    @functools.partial(
        pl.kernel,
        mesh=_mesh,
        compiler_params=_sc_params,
        out_type=jax.ShapeDtypeStruct((NCORES, C, NPAD), f32),
        scratch_types=[
            pltpu.VMEM((1, EB), i32),
            pltpu.VMEM((1, EB), i32),
            pltpu.VMEM((1, EB), i32),
            pltpu.VMEM((1, EB), i32),
            pltpu.VMEM((C, EB), f32),
            pltpu.VMEM((C, EB), f32),
            pltpu.VMEM((SLICE,), f32),
            pltpu.VMEM_SHARED((C, NPAD), f32),
            pltpu.SemaphoreType.DMA,
            pltpu.SemaphoreType.DMA,
        ],
    )
    def _msg(src_hbm, dst_hbm, tab_hbm, agg_hbm,
             sbuf0, sbuf1, dbuf0, dbuf1, gv0, gv1, bounce, agg_sh,
             gsem0, gsem1):
        c = lax.axis_index("c")
        s = lax.axis_index("s")
        wid = c * NSUB + s
        sbuf = (sbuf0, sbuf1)
        dbuf = (dbuf0, dbuf1)
        gv = (gv0, gv1)
        gsem = (gsem0, gsem1)

        sl = pl.ds(s * SLICE, SLICE)

        @pl.loop(0, SLICE, step=16)
        def _(i):
            bounce[pl.ds(i, 16)] = jnp.zeros((16,), f32)

        for ch in range(C):
            pltpu.sync_copy(bounce, agg_sh.at[ch].at[sl])
        plsc.subcore_barrier()

        def issue_gathers(k, b):
            pltpu.sync_copy(src_hbm.at[pl.ds(b, 1)], sbuf[k])
            pltpu.sync_copy(dst_hbm.at[pl.ds(b, 1)], dbuf[k])
            for ch in range(C):
                pltpu.make_async_copy(
                    tab_hbm.at[ch].at[sbuf[k].at[0]], gv[k].at[ch], gsem[k]
                ).start()

        def wait_and_scatter(k):
            for ch in range(C):
                pltpu.make_async_copy(
                    tab_hbm.at[ch].at[sbuf[k].at[0]], gv[k].at[ch], gsem[k]
                ).wait()
            for ch in range(C):
                pltpu.sync_copy(gv[k].at[ch], agg_sh.at[ch].at[dbuf[k].at[0]],
                                add=True)

        @pl.loop(0, NB_PER_W // 2)
        def _(th):
            for k in (0, 1):
                i = th * 2 + k

                @pl.when((i >= 1) & (wid + (i - 1) * NW < NB))
                def _():
                    wait_and_scatter(1 - k)

                b = wid + i * NW

                @pl.when(b < NB)
                def _():
                    issue_gathers(k, b)

        @pl.when(wid < NB - (NB_PER_W - 1) * NW)
        def _():
            wait_and_scatter(1)

        plsc.subcore_barrier()
        for ch in range(C):
            pltpu.sync_copy(agg_sh.at[ch].at[sl], bounce)
            pltpu.sync_copy(bounce, agg_hbm.at[c].at[ch].at[sl])

    return _msg


_sc_msg4 = _make_msg_pass(4)
_sc_msg3 = _make_msg_pass(3)


_BR = 6272


def _t1_body(dego_ref, degi_ref, feat_ref, h_ref, rso_ref, rsi_ref):
    dego = dego_ref[0, :] + dego_ref[1, :]
    degi = degi_ref[0, :] + degi_ref[1, :]
    rso = lax.rsqrt(jnp.maximum(dego, 1.0))
    rsi = lax.rsqrt(jnp.maximum(degi, 1.0))
    h_ref[...] = feat_ref[...].T * rso[None, :]
    rso_ref[...] = rso[:, None]
    rsi_ref[...] = rsi[:, None]


def _t1(dego_p, degi_p, featp):
    grid = NPAD // _BR
    return pl.pallas_call(
        _t1_body,
        grid=(grid,),
        in_specs=[
            pl.BlockSpec((NCORES, _BR), lambda i: (0, i)),
            pl.BlockSpec((NCORES, _BR), lambda i: (0, i)),
            pl.BlockSpec((_BR, 4), lambda i: (i, 0)),
        ],
        out_specs=[
            pl.BlockSpec((4, _BR), lambda i: (0, i)),
            pl.BlockSpec((_BR, 1), lambda i: (i, 0)),
            pl.BlockSpec((_BR, 1), lambda i: (i, 0)),
        ],
        out_shape=[
            jax.ShapeDtypeStruct((4, NPAD), f32),
            jax.ShapeDtypeStruct((NPAD, 1), f32),
            jax.ShapeDtypeStruct((NPAD, 1), f32),
        ],
    )(dego_p, degi_p, featp)


def _t2_body(agg_ref, rsi_ref, rso_ref, w1_ref, b1_ref, w2_ref, z_ref):
    a = (agg_ref[0] + agg_ref[1]).T * rsi_ref[...]
    y = jnp.dot(a, w1_ref[...], preferred_element_type=f32) + b1_ref[...]
    y = jnp.where(y >= 0.0, y, 0.01 * y)
    y = y * rso_ref[...]
    z = jnp.dot(y, w2_ref[...], preferred_element_type=f32)
    z_ref[...] = z.T


def _t2(agg1_p, rsi, rso, W1, b1, W2):
    grid = NPAD // _BR
    return pl.pallas_call(
        _t2_body,
        grid=(grid,),
        in_specs=[
            pl.BlockSpec((NCORES, 4, _BR), lambda i: (0, 0, i)),
            pl.BlockSpec((_BR, 1), lambda i: (i, 0)),
            pl.BlockSpec((_BR, 1), lambda i: (i, 0)),
            pl.BlockSpec((4, 64), lambda i: (0, 0)),
            pl.BlockSpec((1, 64), lambda i: (0, 0)),
            pl.BlockSpec((64, 3), lambda i: (0, 0)),
        ],
        out_specs=pl.BlockSpec((3, _BR), lambda i: (0, i)),
        out_shape=jax.ShapeDtypeStruct((3, NPAD), f32),
    )(agg1_p, rsi, rso, W1, b1.reshape(1, 64), W2)


def _t3_body(agg_ref, rsi_ref, b2_ref, out_ref):
    a = (agg_ref[0] + agg_ref[1]).T
    out_ref[...] = a * rsi_ref[...] + b2_ref[...]


def _t3(agg2_p, rsi, b2):
    grid = NPAD // _BR
    return pl.pallas_call(
        _t3_body,
        grid=(grid,),
        in_specs=[
            pl.BlockSpec((NCORES, 3, _BR), lambda i: (0, 0, i)),
            pl.BlockSpec((_BR, 1), lambda i: (i, 0)),
            pl.BlockSpec((1, 3), lambda i: (0, 0)),
        ],
        out_specs=pl.BlockSpec((_BR, 3), lambda i: (i, 0)),
        out_shape=jax.ShapeDtypeStruct((NPAD, 3), f32),
    )(agg2_p, rsi, b2.reshape(1, 3))


def kernel(features, edge_index, W1, b1, W2, b2):
    src = edge_index[0].astype(i32).reshape(NB, EB)
    dst = edge_index[1].astype(i32).reshape(NB, EB)

    dego_p, degi_p = _sc_degrees(src, dst)

    featp = jnp.zeros((NPAD, 4), f32).at[:N_NODES_C].set(features)
    h1, rso, rsi = _t1(dego_p, degi_p, featp)

    agg1_p = _sc_msg4(src, dst, h1)
    z3 = _t2(agg1_p, rsi, rso, W1, b1, W2)
    agg2_p = _sc_msg3(src, dst, z3)
    out = _t3(agg2_p, rsi, b2)
    return out[:N_NODES_C]


# speedup vs baseline: 1.2430x; 1.2430x over previous
import functools

import jax
import jax.numpy as jnp
from jax import lax
from jax.experimental import pallas as pl
from jax.experimental.pallas import tpu as pltpu
from jax.experimental.pallas import tpu_sc as plsc

N_NODES_C = 100000
N_EDGES_C = 6400000

NCORES = 2
NSUB = 16
NW = NCORES * NSUB

EB = 2048
NB = N_EDGES_C // EB
NB_PER_W = -(-NB // NW)

NPAD = 100352
SLICE = NPAD // NSUB

_mesh = plsc.VectorSubcoreMesh(core_axis_name="c", subcore_axis_name="s")
_sc_params = pltpu.CompilerParams(use_tc_tiling_on_sc=False)

f32 = jnp.float32
i32 = jnp.int32


@functools.partial(
    pl.kernel,
    mesh=_mesh,
    compiler_params=_sc_params,
    out_type=[
        jax.ShapeDtypeStruct((NCORES, NPAD), f32),
        jax.ShapeDtypeStruct((NCORES, NPAD), f32),
    ],
    scratch_types=[
        pltpu.VMEM((1, EB), i32),
        pltpu.VMEM((1, EB), i32),
        pltpu.VMEM((EB,), f32),
        pltpu.VMEM((SLICE,), f32),
        pltpu.VMEM_SHARED((NPAD,), f32),
        pltpu.VMEM_SHARED((NPAD,), f32),
    ],
)
def _sc_degrees(src_hbm, dst_hbm, dego_hbm, degi_hbm,
                sbuf, dbuf, ones_v, bounce, dego_sh, degi_sh):
    c = lax.axis_index("c")
    s = lax.axis_index("s")
    wid = c * NSUB + s

    @pl.loop(0, EB, step=16)
    def _(i):
        ones_v[pl.ds(i, 16)] = jnp.ones((16,), f32)

    @pl.loop(0, SLICE, step=16)
    def _(i):
        bounce[pl.ds(i, 16)] = jnp.zeros((16,), f32)

    sl = pl.ds(s * SLICE, SLICE)
    pltpu.sync_copy(bounce, dego_sh.at[sl])
    pltpu.sync_copy(bounce, degi_sh.at[sl])
    plsc.subcore_barrier()

    @pl.loop(0, NB_PER_W)
    def _(i):
        b = wid + i * NW

        @pl.when(b < NB)
        def _():
            pltpu.sync_copy(src_hbm.at[pl.ds(b, 1)], sbuf)
            pltpu.sync_copy(dst_hbm.at[pl.ds(b, 1)], dbuf)
            pltpu.sync_copy(ones_v, dego_sh.at[sbuf.at[0]], add=True)
            pltpu.sync_copy(ones_v, degi_sh.at[dbuf.at[0]], add=True)

    plsc.subcore_barrier()
    pltpu.sync_copy(dego_sh.at[sl], bounce)
    pltpu.sync_copy(bounce, dego_hbm.at[c].at[sl])
    pltpu.sync_copy(degi_sh.at[sl], bounce)
    pltpu.sync_copy(bounce, degi_hbm.at[c].at[sl])


def _make_msg_pass(C, G):
    @functools.partial(
        pl.kernel,
        mesh=_mesh,
        compiler_params=_sc_params,
        out_type=jax.ShapeDtypeStruct((NCORES, C, NPAD), f32),
        scratch_types=[
            pltpu.VMEM((1, EB), i32),
            pltpu.VMEM((1, EB), i32),
            pltpu.VMEM((1, EB), i32),
            pltpu.VMEM((1, EB), i32),
            pltpu.VMEM((C, EB), f32),
            pltpu.VMEM((C, EB), f32),
            pltpu.VMEM((SLICE,), f32),
            pltpu.VMEM_SHARED((C, NPAD), f32),
            pltpu.VMEM_SHARED((C, NPAD), f32),
            pltpu.SemaphoreType.DMA,
            pltpu.SemaphoreType.DMA,
        ],
    )
    def _msg(src_hbm, dst_hbm, tab_hbm, agg_hbm,
             sbuf0, sbuf1, dbuf0, dbuf1, gv0, gv1, bounce, tab_sh, agg_sh,
             gsem0, gsem1):
        c = lax.axis_index("c")
        s = lax.axis_index("s")
        wid = c * NSUB + s
        sbuf = (sbuf0, sbuf1)
        dbuf = (dbuf0, dbuf1)
        gv = (gv0, gv1)
        gsem = (gsem0, gsem1)

        sl = pl.ds(s * SLICE, SLICE)

        @pl.loop(0, SLICE, step=16)
        def _(i):
            bounce[pl.ds(i, 16)] = jnp.zeros((16,), f32)

        for ch in range(C):
            pltpu.sync_copy(bounce, agg_sh.at[ch].at[sl])
        for ch in range(G, C):
            pltpu.sync_copy(tab_hbm.at[ch].at[sl], bounce)
            pltpu.sync_copy(bounce, tab_sh.at[ch].at[sl])
        plsc.subcore_barrier()

        def issue_gathers(k, b):
            pltpu.sync_copy(src_hbm.at[pl.ds(b, 1)], sbuf[k])
            pltpu.sync_copy(dst_hbm.at[pl.ds(b, 1)], dbuf[k])
            for ch in range(G):
                pltpu.make_async_copy(
                    tab_hbm.at[ch].at[sbuf[k].at[0]], gv[k].at[ch], gsem[k]
                ).start()

        def wait_and_scatter(k):
            for ch in range(G):
                pltpu.make_async_copy(
                    tab_hbm.at[ch].at[sbuf[k].at[0]], gv[k].at[ch], gsem[k]
                ).wait()
            for ch in range(G, C):
                pltpu.sync_copy(tab_sh.at[ch].at[sbuf[k].at[0]], gv[k].at[ch])
            for ch in range(C):
                pltpu.sync_copy(gv[k].at[ch], agg_sh.at[ch].at[dbuf[k].at[0]],
                                add=True)

        @pl.loop(0, NB_PER_W // 2)
        def _(th):
            for k in (0, 1):
                i = th * 2 + k

                @pl.when((i >= 1) & (wid + (i - 1) * NW < NB))
                def _():
                    wait_and_scatter(1 - k)

                b = wid + i * NW

                @pl.when(b < NB)
                def _():
                    issue_gathers(k, b)

        @pl.when(wid < NB - (NB_PER_W - 1) * NW)
        def _():
            wait_and_scatter(1)

        plsc.subcore_barrier()
        for ch in range(C):
            pltpu.sync_copy(agg_sh.at[ch].at[sl], bounce)
            pltpu.sync_copy(bounce, agg_hbm.at[c].at[ch].at[sl])

    return _msg


_sc_msg4 = _make_msg_pass(4, 2)
_sc_msg3 = _make_msg_pass(3, 1)


_BR = 6272


def _t1_body(dego_ref, degi_ref, feat_ref, h_ref, rso_ref, rsi_ref):
    dego = dego_ref[0, :] + dego_ref[1, :]
    degi = degi_ref[0, :] + degi_ref[1, :]
    rso = lax.rsqrt(jnp.maximum(dego, 1.0))
    rsi = lax.rsqrt(jnp.maximum(degi, 1.0))
    h_ref[...] = feat_ref[...].T * rso[None, :]
    rso_ref[...] = rso[:, None]
    rsi_ref[...] = rsi[:, None]


def _t1(dego_p, degi_p, featp):
    grid = NPAD // _BR
    return pl.pallas_call(
        _t1_body,
        grid=(grid,),
        in_specs=[
            pl.BlockSpec((NCORES, _BR), lambda i: (0, i)),
            pl.BlockSpec((NCORES, _BR), lambda i: (0, i)),
            pl.BlockSpec((_BR, 4), lambda i: (i, 0)),
        ],
        out_specs=[
            pl.BlockSpec((4, _BR), lambda i: (0, i)),
            pl.BlockSpec((_BR, 1), lambda i: (i, 0)),
            pl.BlockSpec((_BR, 1), lambda i: (i, 0)),
        ],
        out_shape=[
            jax.ShapeDtypeStruct((4, NPAD), f32),
            jax.ShapeDtypeStruct((NPAD, 1), f32),
            jax.ShapeDtypeStruct((NPAD, 1), f32),
        ],
    )(dego_p, degi_p, featp)


def _t2_body(agg_ref, rsi_ref, rso_ref, w1_ref, b1_ref, w2_ref, z_ref):
    a = (agg_ref[0] + agg_ref[1]).T * rsi_ref[...]
    y = jnp.dot(a, w1_ref[...], preferred_element_type=f32) + b1_ref[...]
    y = jnp.where(y >= 0.0, y, 0.01 * y)
    y = y * rso_ref[...]
    z = jnp.dot(y, w2_ref[...], preferred_element_type=f32)
    z_ref[...] = z.T


def _t2(agg1_p, rsi, rso, W1, b1, W2):
    grid = NPAD // _BR
    return pl.pallas_call(
        _t2_body,
        grid=(grid,),
        in_specs=[
            pl.BlockSpec((NCORES, 4, _BR), lambda i: (0, 0, i)),
            pl.BlockSpec((_BR, 1), lambda i: (i, 0)),
            pl.BlockSpec((_BR, 1), lambda i: (i, 0)),
            pl.BlockSpec((4, 64), lambda i: (0, 0)),
            pl.BlockSpec((1, 64), lambda i: (0, 0)),
            pl.BlockSpec((64, 3), lambda i: (0, 0)),
        ],
        out_specs=pl.BlockSpec((3, _BR), lambda i: (0, i)),
        out_shape=jax.ShapeDtypeStruct((3, NPAD), f32),
    )(agg1_p, rsi, rso, W1, b1.reshape(1, 64), W2)


def _t3_body(agg_ref, rsi_ref, b2_ref, out_ref):
    a = (agg_ref[0] + agg_ref[1]).T
    out_ref[...] = a * rsi_ref[...] + b2_ref[...]


def _t3(agg2_p, rsi, b2):
    grid = NPAD // _BR
    return pl.pallas_call(
        _t3_body,
        grid=(grid,),
        in_specs=[
            pl.BlockSpec((NCORES, 3, _BR), lambda i: (0, 0, i)),
            pl.BlockSpec((_BR, 1), lambda i: (i, 0)),
            pl.BlockSpec((1, 3), lambda i: (0, 0)),
        ],
        out_specs=pl.BlockSpec((_BR, 3), lambda i: (i, 0)),
        out_shape=jax.ShapeDtypeStruct((NPAD, 3), f32),
    )(agg2_p, rsi, b2.reshape(1, 3))


def kernel(features, edge_index, W1, b1, W2, b2):
    src = edge_index[0].astype(i32).reshape(NB, EB)
    dst = edge_index[1].astype(i32).reshape(NB, EB)

    dego_p, degi_p = _sc_degrees(src, dst)

    featp = jnp.zeros((NPAD, 4), f32).at[:N_NODES_C].set(features)
    h1, rso, rsi = _t1(dego_p, degi_p, featp)

    agg1_p = _sc_msg4(src, dst, h1)
    z3 = _t2(agg1_p, rsi, rso, W1, b1, W2)
    agg2_p = _sc_msg3(src, dst, z3)
    out = _t3(agg2_p, rsi, b2)
    return out[:N_NODES_C]
